# baseline (speedup 1.0000x reference)
import jax
import jax.numpy as jnp
from jax.experimental import pallas as pl
from jax.experimental.pallas import tpu as pltpu

B, S, H = 2, 4096, 768
FF, R, E, TOPK = 3072, 32, 8, 2
ER = E * R
TILE = 1024


def _moe_ffn_kernel(x_ref, wrt_ref, br_ref, w1s_ref, w2s_ref, wl2t_ref,
                    bl2_ref, out_ref):
    x = x_ref[...]

    logits = jnp.dot(x, wrt_ref[...], preferred_element_type=jnp.float32)
    logits = logits + br_ref[...]
    m = jnp.max(logits, axis=-1, keepdims=True)
    ex = jnp.exp(logits - m)
    w = ex / jnp.sum(ex, axis=-1, keepdims=True)

    lane = jax.lax.broadcasted_iota(jnp.int32, w.shape, 1)
    m1 = jnp.max(w, axis=-1, keepdims=True)
    idx1 = jnp.min(jnp.where(w == m1, lane, E), axis=-1, keepdims=True)
    sel1 = lane == idx1
    w_rest = jnp.where(sel1, -jnp.inf, w)
    m2 = jnp.max(w_rest, axis=-1, keepdims=True)
    idx2 = jnp.min(jnp.where(w_rest == m2, lane, E), axis=-1, keepdims=True)
    gate = jnp.where(sel1 | (lane == idx2), w, 0.0)

    re = jax.lax.broadcasted_iota(jnp.int32, (E, ER), 0)
    ce = jax.lax.broadcasted_iota(jnp.int32, (E, ER), 1)
    expand = (ce // R == re).astype(jnp.float32)
    gate_exp = jnp.dot(gate, expand, preferred_element_type=jnp.float32)

    xb = x.astype(jnp.bfloat16)
    h = jnp.dot(xb, w1s_ref[...], preferred_element_type=jnp.float32)
    hg = (h * gate_exp).astype(jnp.bfloat16)

    nc = 4
    fc = FF // nc
    out = bl2_ref[...]
    for c in range(nc):
        mixed = jnp.dot(hg, w2s_ref[:, c * fc:(c + 1) * fc],
                        preferred_element_type=jnp.float32)
        act = 0.5 * mixed * (1.0 + jax.lax.erf(mixed * 0.7071067811865476))
        out = out + jnp.dot(act.astype(jnp.bfloat16),
                            wl2t_ref[c * fc:(c + 1) * fc, :],
                            preferred_element_type=jnp.float32)
    out_ref[...] = out


@jax.jit
def kernel(x, Wr, br, W1, W2, W_lin2, b_lin2):
    xf = x.reshape(B * S, H)
    wrt = Wr.T
    w1s = W1.reshape(ER, H).T.astype(jnp.bfloat16)
    w2s = W2.transpose(0, 2, 1).reshape(ER, FF).astype(jnp.bfloat16)
    wl2t = W_lin2.T.astype(jnp.bfloat16)
    br2 = br.reshape(1, E)
    bl2 = b_lin2.reshape(1, H)

    n_tiles = (B * S) // TILE
    out = pl.pallas_call(
        _moe_ffn_kernel,
        grid=(n_tiles,),
        in_specs=[
            pl.BlockSpec((TILE, H), lambda i: (i, 0)),
            pl.BlockSpec((H, E), lambda i: (0, 0)),
            pl.BlockSpec((1, E), lambda i: (0, 0)),
            pl.BlockSpec((H, ER), lambda i: (0, 0)),
            pl.BlockSpec((ER, FF), lambda i: (0, 0)),
            pl.BlockSpec((FF, H), lambda i: (0, 0)),
            pl.BlockSpec((1, H), lambda i: (0, 0)),
        ],
        out_specs=pl.BlockSpec((TILE, H), lambda i: (i, 0)),
        out_shape=jax.ShapeDtypeStruct((B * S, H), jnp.float32),
        compiler_params=pltpu.CompilerParams(
            dimension_semantics=("parallel",)),
    )(xf, wrt, br2, w1s, w2s, wl2t, bl2)
    return out.reshape(B, S, H)


# speedup vs baseline: 1.0804x; 1.0804x over previous
import jax
import jax.numpy as jnp
from jax.experimental import pallas as pl
from jax.experimental.pallas import tpu as pltpu

B, S, H = 2, 4096, 768
FF, R, E, TOPK = 3072, 32, 8, 2
ER = E * R
TILE = 1024


def _moe_ffn_kernel(x_ref, wrt_ref, br_ref, w1r_ref, w2s_ref, wl2_ref,
                    bl2_ref, out_ref, w1s_s, wl2t_s):
    @pl.when(pl.program_id(0) == 0)
    def _prep():
        w1s_s[...] = w1r_ref[...].T.astype(jnp.bfloat16)
        wl2t_s[...] = wl2_ref[...].T.astype(jnp.bfloat16)

    x = x_ref[...]

    logits = jnp.dot(x, wrt_ref[...], preferred_element_type=jnp.float32)
    logits = logits + br_ref[...]
    m = jnp.max(logits, axis=-1, keepdims=True)
    ex = jnp.exp(logits - m)
    w = ex / jnp.sum(ex, axis=-1, keepdims=True)

    lane = jax.lax.broadcasted_iota(jnp.int32, w.shape, 1)
    m1 = jnp.max(w, axis=-1, keepdims=True)
    idx1 = jnp.min(jnp.where(w == m1, lane, E), axis=-1, keepdims=True)
    sel1 = lane == idx1
    w_rest = jnp.where(sel1, -jnp.inf, w)
    m2 = jnp.max(w_rest, axis=-1, keepdims=True)
    idx2 = jnp.min(jnp.where(w_rest == m2, lane, E), axis=-1, keepdims=True)
    gate = jnp.where(sel1 | (lane == idx2), w, 0.0)

    re = jax.lax.broadcasted_iota(jnp.int32, (E, ER), 0)
    ce = jax.lax.broadcasted_iota(jnp.int32, (E, ER), 1)
    expand = (ce // R == re).astype(jnp.float32)
    gate_exp = jnp.dot(gate, expand, preferred_element_type=jnp.float32)

    xb = x.astype(jnp.bfloat16)
    h = jnp.dot(xb, w1s_s[...], preferred_element_type=jnp.float32)
    hg = (h * gate_exp).astype(jnp.bfloat16)

    nc = 4
    fc = FF // nc
    out = bl2_ref[...]
    for c in range(nc):
        mixed = jnp.dot(hg, w2s_ref[:, c * fc:(c + 1) * fc],
                        preferred_element_type=jnp.float32)
        act = 0.5 * mixed * (1.0 + jax.lax.erf(mixed * 0.7071067811865476))
        out = out + jnp.dot(act.astype(jnp.bfloat16),
                            wl2t_s[c * fc:(c + 1) * fc, :],
                            preferred_element_type=jnp.float32)
    out_ref[...] = out


@jax.jit
def kernel(x, Wr, br, W1, W2, W_lin2, b_lin2):
    xf = x.reshape(B * S, H)
    wrt = Wr.T
    w1r = W1.reshape(ER, H)
    w2s = W2.transpose(0, 2, 1).reshape(ER, FF).astype(jnp.bfloat16)
    br2 = br.reshape(1, E)
    bl2 = b_lin2.reshape(1, H)

    n_tiles = (B * S) // TILE
    out = pl.pallas_call(
        _moe_ffn_kernel,
        grid=(n_tiles,),
        in_specs=[
            pl.BlockSpec((TILE, H), lambda i: (i, 0)),
            pl.BlockSpec((H, E), lambda i: (0, 0)),
            pl.BlockSpec((1, E), lambda i: (0, 0)),
            pl.BlockSpec((ER, H), lambda i: (0, 0)),
            pl.BlockSpec((ER, FF), lambda i: (0, 0)),
            pl.BlockSpec((H, FF), lambda i: (0, 0)),
            pl.BlockSpec((1, H), lambda i: (0, 0)),
        ],
        out_specs=pl.BlockSpec((TILE, H), lambda i: (i, 0)),
        out_shape=jax.ShapeDtypeStruct((B * S, H), jnp.float32),
        scratch_shapes=[
            pltpu.VMEM((H, ER), jnp.bfloat16),
            pltpu.VMEM((FF, H), jnp.bfloat16),
        ],
    )(xf, wrt, br2, w1r, w2s, W_lin2, bl2)
    return out.reshape(B, S, H)
